# baseline (speedup 1.0000x reference)
import functools

import jax
import jax.numpy as jnp
from jax import lax
from jax.experimental import pallas as pl
from jax.experimental.pallas import tpu as pltpu
from jax.experimental.pallas import tpu_sc as plsc

NUM_NODES = 10000
HID = 64
ALPHA = 0.1
ITERS = 10

NC = 2
NS = 16
CHUNK = 128

NPAD = 10240
ROWS_PER_TILE = NPAD // NS
ZR = 128

_SC_PARAMS = pltpu.CompilerParams(needs_layout_passes=False,
                                  use_tc_tiling_on_sc=False)


def _splat(ref2d, i, j):
  ii = jnp.full((16,), i, jnp.int32)
  jj = jnp.full((16,), j, jnp.int32)
  return plsc.load_gather(ref2d, [ii, jj])


def _scale_rows(gbuf, w_ref, ch, n_edges, width, factor=1.0):
  @plsc.parallel_loop(0, n_edges, unroll=16)
  def body(j):
    w = _splat(w_ref, ch, j)
    if factor != 1.0:
      w = w * factor
    for k in range(width // 16):
      gbuf[j, pl.ds(16 * k, 16)] = gbuf[j, pl.ds(16 * k, 16)] * w


def _scatter_add_rows(shared, idx_ref, src):
  pltpu.sync_copy(src, shared.at[idx_ref], add=True)


def _zero_rows(shared, zbuf, row0, nrows):
  def zb(i, _):
    zeros = jnp.zeros((16,), jnp.float32)
    for k in range(HID // 16):
      zbuf[i, pl.ds(16 * k, 16)] = zeros
    return 0
  lax.fori_loop(0, ZR, zb, 0)
  for r in range(nrows // ZR):
    pltpu.sync_copy(zbuf, shared.at[pl.ds(row0 + ZR * r, ZR)])


def _edge_phase(hbm3, src, dst, slots, ebase, nchunks, width, factor):
  def edges_io(b, ch, issue):
    off = ebase + ch * CHUNK
    sl = slots[b]
    for h, r in zip(hbm3, (sl["gi"], sl["si"], sl["vw"])):
      if issue:
        pltpu.async_copy(h.at[pl.ds(off, CHUNK)], r.at[0], sl["esem"])
      else:
        pltpu.make_async_copy(h.at[pl.ds(off, CHUNK)], r.at[0],
                              sl["esem"]).wait()

  def gather(b, issue):
    sl = slots[b]
    if issue:
      pltpu.async_copy(src.at[sl["gi"].at[0]], sl["gbuf"], sl["gsem"])
    else:
      pltpu.make_async_copy(src.at[sl["gi"].at[0]], sl["gbuf"],
                            sl["gsem"]).wait()

  def scatter(b, issue):
    sl = slots[b]
    if issue:
      pltpu.async_copy(sl["gbuf"], dst.at[sl["si"].at[0]], sl["ssem"],
                       add=True)
    else:
      pltpu.make_async_copy(sl["gbuf"], dst.at[sl["si"].at[0]],
                            sl["ssem"]).wait()

  def step(ch, b, has_next, has_next2, has_prev):
    b1, b2 = (b + 1) % 3, (b + 2) % 3
    gather(b, False)
    if has_next:
      edges_io(b1, ch + 1, False)
      gather(b1, True)
    _scale_rows(slots[b]["gbuf"], slots[b]["vw"], 0, CHUNK, width, factor)
    scatter(b, True)
    if has_prev:
      scatter(b2, False)
    if has_next2:
      edges_io(b2, ch + 2, True)

  edges_io(0, 0, True)
  edges_io(1, 1, True)
  edges_io(0, 0, False)
  gather(0, True)
  step(0, 0, True, True, False)
  step(1, 1, True, True, True)

  def steady(i, _):
    for o, b in ((2, 2), (3, 0), (4, 1)):
      step(3 * i + o, b, True, True, True)
    return 0

  lax.fori_loop(0, (nchunks - 4) // 3, steady, 0)

  step(nchunks - 2, (nchunks - 2) % 3, True, False, True)
  step(nchunks - 1, (nchunks - 1) % 3, False, False, True)
  scatter((nchunks - 1) % 3, False)


def _edge_slot_scratch(width):
  return [
      pltpu.VMEM((1, CHUNK), jnp.int32),
      pltpu.VMEM((1, CHUNK), jnp.int32),
      pltpu.VMEM((1, CHUNK), jnp.float32),
      pltpu.VMEM((CHUNK, width), jnp.float32),
      pltpu.SemaphoreType.DMA,
      pltpu.SemaphoreType.DMA,
      pltpu.SemaphoreType.DMA,
  ]


def _make_slots(refs):
  keys = ("gi", "si", "vw", "gbuf", "esem", "gsem", "ssem")
  n = len(keys)
  return [dict(zip(keys, refs[i * n:(i + 1) * n])) for i in range(3)]


def _feature_spmm(frow, fcol, fval, W1):
  nnz = frow.shape[0]
  chunks_per_tile = nnz // (NC * NS * CHUNK)
  per_tile = chunks_per_tile * CHUNK

  mesh = plsc.VectorSubcoreMesh(core_axis_name="c", subcore_axis_name="s",
                                num_cores=NC, num_subcores=NS)

  @functools.partial(
      pl.kernel,
      out_type=jax.ShapeDtypeStruct((NC, NPAD, HID), jnp.float32),
      mesh=mesh,
      scratch_types=[
          pltpu.VMEM_SHARED((NPAD, HID), jnp.float32),
          pltpu.VMEM_SHARED((NPAD, HID), jnp.float32),
          pltpu.VMEM((ZR, HID), jnp.float32),
      ] + _edge_slot_scratch(HID) * 3,
      compiler_params=_SC_PARAMS,
  )
  def k(frow_h, fcol_h, fval_h, w1_h, out_h, w1_s, acc, zbuf, *slot_refs):
    c = lax.axis_index("c")
    s = lax.axis_index("s")
    slots = _make_slots(list(slot_refs))
    row0 = s * ROWS_PER_TILE
    pltpu.sync_copy(w1_h.at[pl.ds(row0, ROWS_PER_TILE)],
                    w1_s.at[pl.ds(row0, ROWS_PER_TILE)])
    _zero_rows(acc, zbuf, row0, ROWS_PER_TILE)
    plsc.subcore_barrier()

    ebase = (c * NS + s) * per_tile
    _edge_phase((fcol_h, frow_h, fval_h), w1_s, acc, slots, ebase,
                chunks_per_tile, HID, 1.0)
    plsc.subcore_barrier()
    pltpu.sync_copy(acc.at[pl.ds(row0, ROWS_PER_TILE)],
                    out_h.at[c, pl.ds(row0, ROWS_PER_TILE)])

  return k(frow, fcol, fval, W1)


def _dense_tc(hpart, b1, W2, b2):
  blk = 1024

  H2 = HID // 2

  def body(hp_ref, b1_ref, w2_ref, b2_ref, h2_ref, t2_ref):
    h = hp_ref[0] + hp_ref[1] + b1_ref[...][None, :]
    h = jnp.maximum(h, 0.0)
    h2 = (jnp.dot(h, w2_ref[...], preferred_element_type=jnp.float32)
          + b2_ref[...][None, :])
    h2_ref[0] = h2[:, :H2]
    h2_ref[1] = h2[:, H2:]
    t2_ref[0] = ALPHA * h2[:, :H2]
    t2_ref[1] = ALPHA * h2[:, H2:]

  return pl.pallas_call(
      body,
      out_shape=[jax.ShapeDtypeStruct((NC, NPAD, H2), jnp.float32),
                 jax.ShapeDtypeStruct((NC, NPAD, H2), jnp.float32)],
      grid=(NPAD // blk,),
      in_specs=[
          pl.BlockSpec((NC, blk, HID), lambda i: (0, i, 0)),
          pl.BlockSpec((HID,), lambda i: (0,)),
          pl.BlockSpec((HID, HID), lambda i: (0, 0)),
          pl.BlockSpec((HID,), lambda i: (0,)),
      ],
      out_specs=[pl.BlockSpec((NC, blk, H2), lambda i: (0, i, 0)),
                 pl.BlockSpec((NC, blk, H2), lambda i: (0, i, 0))],
  )(hpart, b1, W2, b2)


def _appnp(h2, t2, erow, ecol, ew):
  n_edges = erow.shape[0]
  chunks_per_tile = n_edges // (NS * CHUNK)
  per_tile = chunks_per_tile * CHUNK
  H2 = HID // 2

  mesh = plsc.VectorSubcoreMesh(core_axis_name="c", subcore_axis_name="s",
                                num_cores=NC, num_subcores=NS)

  @functools.partial(
      pl.kernel,
      out_type=jax.ShapeDtypeStruct((NC, NPAD, H2), jnp.float32),
      mesh=mesh,
      scratch_types=[
          pltpu.VMEM_SHARED((NPAD, H2), jnp.float32),
          pltpu.VMEM_SHARED((NPAD, H2), jnp.float32),
      ] + _edge_slot_scratch(HID // 2) * 3,
      compiler_params=_SC_PARAMS,
  )
  def k(h2_h, t2_h, erow_h, ecol_h, ew_h, out_h, A, B, *slot_refs):
    c = lax.axis_index("c")
    s = lax.axis_index("s")
    slots = _make_slots(list(slot_refs))

    row0 = s * ROWS_PER_TILE
    pltpu.sync_copy(h2_h.at[c, pl.ds(row0, ROWS_PER_TILE)],
                    A.at[pl.ds(row0, ROWS_PER_TILE)])
    ebase = s * per_tile
    plsc.subcore_barrier()

    def one_iter(src, dst):
      pltpu.sync_copy(t2_h.at[c, pl.ds(row0, ROWS_PER_TILE)],
                      dst.at[pl.ds(row0, ROWS_PER_TILE)])
      plsc.subcore_barrier()
      _edge_phase((ecol_h, erow_h, ew_h), src, dst, slots, ebase,
                  chunks_per_tile, H2, 1.0 - ALPHA)
      plsc.subcore_barrier()

    def iter_pair(_, carry):
      one_iter(A, B)
      one_iter(B, A)
      return carry

    lax.fori_loop(0, ITERS // 2, iter_pair, 0)
    pltpu.sync_copy(A.at[pl.ds(row0, ROWS_PER_TILE)],
                    out_h.at[c, pl.ds(row0, ROWS_PER_TILE)])

  return k(h2, t2, erow, ecol, ew)


def _log_softmax_tc(halves):
  blk = 1024
  H2 = HID // 2

  def body(x_ref, out_ref):
    v = jnp.concatenate([x_ref[0], x_ref[1]], axis=1)
    m = jnp.max(v, axis=1, keepdims=True)
    e = jnp.exp(v - m)
    out_ref[...] = (v - m) - jnp.log(jnp.sum(e, axis=1, keepdims=True))

  return pl.pallas_call(
      body,
      out_shape=jax.ShapeDtypeStruct((NPAD, HID), jnp.float32),
      grid=(NPAD // blk,),
      in_specs=[pl.BlockSpec((NC, blk, H2), lambda i: (0, i, 0))],
      out_specs=pl.BlockSpec((blk, HID), lambda i: (i, 0)),
  )(halves)


def _pad_to(x, n, fill=0):
  pad = n - x.shape[0]
  return jnp.pad(x, (0, pad), constant_values=fill)


def kernel(feature_indices, feature_values, edge_indices, edge_weights,
           W1, b1, W2, b2):
  nnz = feature_values.shape[0]
  n_edges = edge_weights.shape[0]

  def _padded(n, workers):
    q = workers * CHUNK
    chunks = (n + q - 1) // q
    while chunks % 3 != 1 or chunks < 7:
      chunks += 1
    return chunks * q

  nnz_p = _padded(nnz, NC * NS)
  frow = _pad_to(feature_indices[0], nnz_p)
  fcol = _pad_to(feature_indices[1], nnz_p)
  fval = _pad_to(feature_values, nnz_p)

  ne_p = _padded(n_edges, NS)
  erow = _pad_to(edge_indices[0], ne_p)
  ecol = _pad_to(edge_indices[1], ne_p)
  ew = _pad_to(edge_weights, ne_p)

  W1p = jnp.pad(W1, ((0, NPAD - W1.shape[0]), (0, 0)))
  hpart = _feature_spmm(frow, fcol, fval, W1p)
  h2, t2 = _dense_tc(hpart, b1, W2, b2)
  loc = _appnp(h2, t2, erow, ecol, ew)
  return _log_softmax_tc(loc)[:NUM_NODES]


# speedup vs baseline: 1.3945x; 1.3945x over previous
import functools

import jax
import jax.numpy as jnp
from jax import lax
from jax.experimental import pallas as pl
from jax.experimental.pallas import tpu as pltpu
from jax.experimental.pallas import tpu_sc as plsc

NUM_NODES = 10000
HID = 64
ALPHA = 0.1
ITERS = 10

NC = 2
NS = 16
CHUNK = 128


NPAD = 10240
ROWS_PER_TILE = NPAD // NS
ZR = 128

_SC_PARAMS = pltpu.CompilerParams(needs_layout_passes=False,
                                  use_tc_tiling_on_sc=False)


def _splat(ref2d, i, j):
  ii = jnp.full((16,), i, jnp.int32)
  jj = jnp.full((16,), j, jnp.int32)
  return plsc.load_gather(ref2d, [ii, jj])


def _scale_rows(gbuf, w_ref, ch, n_edges, width, factor=1.0):
  @plsc.parallel_loop(0, n_edges, unroll=8)
  def body(j):
    w = _splat(w_ref, ch, j)
    if factor != 1.0:
      w = w * factor
    for k in range(width // 16):
      gbuf[j, pl.ds(16 * k, 16)] = gbuf[j, pl.ds(16 * k, 16)] * w


def _scatter_add_rows(shared, idx_ref, src):
  pltpu.sync_copy(src, shared.at[idx_ref], add=True)


def _zero_rows(shared, zbuf, row0, nrows):
  def zb(i, _):
    zeros = jnp.zeros((16,), jnp.float32)
    for k in range(HID // 16):
      zbuf[i, pl.ds(16 * k, 16)] = zeros
    return 0
  lax.fori_loop(0, ZR, zb, 0)
  for r in range(nrows // ZR):
    pltpu.sync_copy(zbuf, shared.at[pl.ds(row0 + ZR * r, ZR)])


def _edge_phase(hbm3, src, dst, slots, ebase, nchunks, width, factor, sub):
  S = sub * CHUNK
  def edges_io(b, ch, issue):
    off = ebase + ch * S
    sl = slots[b]
    ops = [(hbm3[0].at[pl.ds(off, S)], sl["gi"]),
           (hbm3[2].at[pl.ds(off, S)], sl["vw"].at[0])]
    ops += [(hbm3[1].at[pl.ds(off + k * CHUNK, CHUNK)], sl["si"].at[k])
            for k in range(sub)]
    for hsrc, r in ops:
      if issue:
        pltpu.async_copy(hsrc, r, sl["esem"])
      else:
        pltpu.make_async_copy(hsrc, r, sl["esem"]).wait()

  def gather(b, issue):
    sl = slots[b]
    for k in range(sub):
      idx = sl["gi"].at[pl.ds(k * CHUNK, CHUNK)]
      dstb = sl["gbuf"].at[pl.ds(k * CHUNK, CHUNK)]
      if issue:
        pltpu.async_copy(src.at[idx], dstb, sl["gsem"])
      else:
        pltpu.make_async_copy(src.at[idx], dstb, sl["gsem"]).wait()

  def scatter(b, issue):
    sl = slots[b]
    for k in range(sub):
      srcb = sl["gbuf"].at[pl.ds(k * CHUNK, CHUNK)]
      if issue:
        pltpu.async_copy(srcb, dst.at[sl["si"].at[k]], sl["ssem"], add=True)
      else:
        pltpu.make_async_copy(srcb, dst.at[sl["si"].at[k]],
                              sl["ssem"]).wait()

  def step(ch, b, has_next, has_next2, has_prev):
    b1, b2 = (b + 1) % 3, (b + 2) % 3
    gather(b, False)
    if has_next:
      edges_io(b1, ch + 1, False)
      gather(b1, True)
    _scale_rows(slots[b]["gbuf"], slots[b]["vw"], 0, S, width, factor)
    scatter(b, True)
    if has_prev:
      scatter(b2, False)
    if has_next2:
      edges_io(b2, ch + 2, True)

  edges_io(0, 0, True)
  edges_io(1, 1, True)
  edges_io(0, 0, False)
  gather(0, True)
  step(0, 0, True, True, False)
  step(1, 1, True, True, True)

  def steady(i, _):
    for o, b in ((2, 2), (3, 0), (4, 1)):
      step(3 * i + o, b, True, True, True)
    return 0

  lax.fori_loop(0, (nchunks - 4) // 3, steady, 0)

  step(nchunks - 2, (nchunks - 2) % 3, True, False, True)
  step(nchunks - 1, (nchunks - 1) % 3, False, False, True)
  scatter((nchunks - 1) % 3, False)


def _edge_slot_scratch(width, sub):
  S = sub * CHUNK
  return [
      pltpu.VMEM((S,), jnp.int32),
      pltpu.VMEM((sub, CHUNK), jnp.int32),
      pltpu.VMEM((1, S), jnp.float32),
      pltpu.VMEM((S, width), jnp.float32),
      pltpu.SemaphoreType.DMA,
      pltpu.SemaphoreType.DMA,
      pltpu.SemaphoreType.DMA,
  ]


def _make_slots(refs):
  keys = ("gi", "si", "vw", "gbuf", "esem", "gsem", "ssem")
  n = len(keys)
  return [dict(zip(keys, refs[i * n:(i + 1) * n])) for i in range(3)]


def _feature_spmm(frow, fcol, fval, W1):
  nnz = frow.shape[0]
  chunks_per_tile = nnz // (NC * NS * CHUNK)
  per_tile = chunks_per_tile * CHUNK

  mesh = plsc.VectorSubcoreMesh(core_axis_name="c", subcore_axis_name="s",
                                num_cores=NC, num_subcores=NS)

  @functools.partial(
      pl.kernel,
      out_type=jax.ShapeDtypeStruct((NC, NPAD, HID), jnp.float32),
      mesh=mesh,
      scratch_types=[
          pltpu.VMEM_SHARED((NPAD, HID), jnp.float32),
          pltpu.VMEM_SHARED((NPAD, HID), jnp.float32),
          pltpu.VMEM((ZR, HID), jnp.float32),
      ] + _edge_slot_scratch(HID, 1) * 3,
      compiler_params=_SC_PARAMS,
  )
  def k(frow_h, fcol_h, fval_h, w1_h, out_h, w1_s, acc, zbuf, *slot_refs):
    c = lax.axis_index("c")
    s = lax.axis_index("s")
    slots = _make_slots(list(slot_refs))
    row0 = s * ROWS_PER_TILE
    pltpu.sync_copy(w1_h.at[pl.ds(row0, ROWS_PER_TILE)],
                    w1_s.at[pl.ds(row0, ROWS_PER_TILE)])
    _zero_rows(acc, zbuf, row0, ROWS_PER_TILE)
    plsc.subcore_barrier()

    ebase = (c * NS + s) * per_tile
    _edge_phase((fcol_h, frow_h, fval_h), w1_s, acc, slots, ebase,
                chunks_per_tile, HID, 1.0, 1)
    plsc.subcore_barrier()
    pltpu.sync_copy(acc.at[pl.ds(row0, ROWS_PER_TILE)],
                    out_h.at[c, pl.ds(row0, ROWS_PER_TILE)])

  return k(frow, fcol, fval, W1)


def _dense_tc(hpart, b1, W2, b2):
  blk = 1024

  H2 = HID // 2

  def body(hp_ref, b1_ref, w2_ref, b2_ref, h2_ref, t2_ref):
    h = hp_ref[0] + hp_ref[1] + b1_ref[...][None, :]
    h = jnp.maximum(h, 0.0)
    h2 = (jnp.dot(h, w2_ref[...], preferred_element_type=jnp.float32)
          + b2_ref[...][None, :])
    h2_ref[0] = h2[:, :H2]
    h2_ref[1] = h2[:, H2:]
    t2_ref[0] = ALPHA * h2[:, :H2]
    t2_ref[1] = ALPHA * h2[:, H2:]

  return pl.pallas_call(
      body,
      out_shape=[jax.ShapeDtypeStruct((NC, NPAD, H2), jnp.float32),
                 jax.ShapeDtypeStruct((NC, NPAD, H2), jnp.float32)],
      grid=(NPAD // blk,),
      in_specs=[
          pl.BlockSpec((NC, blk, HID), lambda i: (0, i, 0)),
          pl.BlockSpec((HID,), lambda i: (0,)),
          pl.BlockSpec((HID, HID), lambda i: (0, 0)),
          pl.BlockSpec((HID,), lambda i: (0,)),
      ],
      out_specs=[pl.BlockSpec((NC, blk, H2), lambda i: (0, i, 0)),
                 pl.BlockSpec((NC, blk, H2), lambda i: (0, i, 0))],
  )(hpart, b1, W2, b2)


def _appnp(h2, t2, erow, ecol, ew):
  n_edges = erow.shape[0]
  chunks_per_tile = n_edges // (NS * CHUNK * 4)
  per_tile = chunks_per_tile * CHUNK * 4
  H2 = HID // 2

  mesh = plsc.VectorSubcoreMesh(core_axis_name="c", subcore_axis_name="s",
                                num_cores=NC, num_subcores=NS)

  @functools.partial(
      pl.kernel,
      out_type=jax.ShapeDtypeStruct((NC, NPAD, H2), jnp.float32),
      mesh=mesh,
      scratch_types=[
          pltpu.VMEM_SHARED((NPAD, H2), jnp.float32),
          pltpu.VMEM_SHARED((NPAD, H2), jnp.float32),
      ] + _edge_slot_scratch(HID // 2, 4) * 3,
      compiler_params=_SC_PARAMS,
  )
  def k(h2_h, t2_h, erow_h, ecol_h, ew_h, out_h, A, B, *slot_refs):
    c = lax.axis_index("c")
    s = lax.axis_index("s")
    slots = _make_slots(list(slot_refs))

    row0 = s * ROWS_PER_TILE
    pltpu.sync_copy(h2_h.at[c, pl.ds(row0, ROWS_PER_TILE)],
                    A.at[pl.ds(row0, ROWS_PER_TILE)])
    ebase = s * per_tile
    plsc.subcore_barrier()

    def one_iter(src, dst):
      pltpu.sync_copy(t2_h.at[c, pl.ds(row0, ROWS_PER_TILE)],
                      dst.at[pl.ds(row0, ROWS_PER_TILE)])
      plsc.subcore_barrier()
      _edge_phase((ecol_h, erow_h, ew_h), src, dst, slots, ebase,
                  chunks_per_tile, H2, 1.0 - ALPHA, 4)
      plsc.subcore_barrier()

    def iter_pair(_, carry):
      one_iter(A, B)
      one_iter(B, A)
      return carry

    lax.fori_loop(0, ITERS // 2, iter_pair, 0)
    pltpu.sync_copy(A.at[pl.ds(row0, ROWS_PER_TILE)],
                    out_h.at[c, pl.ds(row0, ROWS_PER_TILE)])

  return k(h2, t2, erow, ecol, ew)


def _log_softmax_tc(halves):
  blk = 1024
  H2 = HID // 2

  def body(x_ref, out_ref):
    v = jnp.concatenate([x_ref[0], x_ref[1]], axis=1)
    m = jnp.max(v, axis=1, keepdims=True)
    e = jnp.exp(v - m)
    out_ref[...] = (v - m) - jnp.log(jnp.sum(e, axis=1, keepdims=True))

  return pl.pallas_call(
      body,
      out_shape=jax.ShapeDtypeStruct((NPAD, HID), jnp.float32),
      grid=(NPAD // blk,),
      in_specs=[pl.BlockSpec((NC, blk, H2), lambda i: (0, i, 0))],
      out_specs=pl.BlockSpec((blk, HID), lambda i: (i, 0)),
  )(halves)


def _pad_to(x, n, fill=0):
  pad = n - x.shape[0]
  return jnp.pad(x, (0, pad), constant_values=fill)


def kernel(feature_indices, feature_values, edge_indices, edge_weights,
           W1, b1, W2, b2):
  nnz = feature_values.shape[0]
  n_edges = edge_weights.shape[0]

  def _padded(n, workers, sub):
    q = workers * CHUNK * sub
    chunks = (n + q - 1) // q
    while chunks % 3 != 1 or chunks < 7:
      chunks += 1
    return chunks * q

  nnz_p = _padded(nnz, NC * NS, 1)
  frow = _pad_to(feature_indices[0], nnz_p)
  fcol = _pad_to(feature_indices[1], nnz_p)
  fval = _pad_to(feature_values, nnz_p)

  ne_p = _padded(n_edges, NS, 4)
  erow = _pad_to(edge_indices[0], ne_p)
  ecol = _pad_to(edge_indices[1], ne_p)
  ew = _pad_to(edge_weights, ne_p)

  W1p = jnp.pad(W1, ((0, NPAD - W1.shape[0]), (0, 0)))
  hpart = _feature_spmm(frow, fcol, fval, W1p)
  h2, t2 = _dense_tc(hpart, b1, W2, b2)
  loc = _appnp(h2, t2, erow, ecol, ew)
  return _log_softmax_tc(loc)[:NUM_NODES]
